# baseline (speedup 1.0000x reference)
import functools

import jax
import jax.numpy as jnp
import numpy as np
from jax import lax
from jax.experimental import pallas as pl
from jax.experimental.pallas import tpu as pltpu
from jax.experimental.pallas import tpu_sc as plsc

_P = 7
_S = 2
_C = 256
_SCALES = (0.25, 0.125, 0.0625, 0.03125)
_HWS = ((200, 200), (100, 100), (50, 50), (25, 25))
_NROWS = sum(h * w for h, w in _HWS)
_NBINS = _P * _P
_NCONTRIB = 16


def _coords(c, limit):
    valid = (c >= -1.0) & (c <= limit)
    c = jnp.clip(c, 0.0, None)
    low = jnp.minimum(jnp.floor(c), limit - 1.0)
    high = jnp.minimum(low + 1.0, limit - 1.0)
    c_adj = jnp.where(low >= limit - 1.0, low, c)
    frac = c_adj - low
    return valid, low.astype(jnp.int32), high.astype(jnp.int32), frac


def _plan(boxes):
    n = boxes.shape[0]
    area = (boxes[:, 2] - boxes[:, 0]) * (boxes[:, 3] - boxes[:, 1])
    s = jnp.sqrt(area)
    lvl = jnp.floor(4.0 + jnp.log2(s / 224.0 + 1e-6))
    li = (jnp.clip(lvl, 2.0, 5.0) - 2.0).astype(jnp.int32)

    scale = jnp.asarray(_SCALES, jnp.float32)[li]
    hf = jnp.asarray([h for h, _ in _HWS], jnp.float32)[li]
    wf = jnp.asarray([w for _, w in _HWS], jnp.float32)[li]
    wi = jnp.asarray([w for _, w in _HWS], jnp.int32)[li]
    offs = [0]
    for h, w in _HWS[:-1]:
        offs.append(offs[-1] + h * w)
    base = jnp.asarray(offs, jnp.int32)[li]

    x1 = boxes[:, 0] * scale
    y1 = boxes[:, 1] * scale
    x2 = boxes[:, 2] * scale
    y2 = boxes[:, 3] * scale
    bin_w = jnp.maximum(x2 - x1, 1.0) / _P
    bin_h = jnp.maximum(y2 - y1, 1.0) / _P
    grid = (jnp.arange(_P, dtype=jnp.float32)[:, None]
            + (jnp.arange(_S, dtype=jnp.float32)[None, :] + 0.5) / _S).reshape(-1)
    ys = y1[:, None] + grid[None, :] * bin_h[:, None]
    xs = x1[:, None] + grid[None, :] * bin_w[:, None]
    vy, yl, yh, ly = _coords(ys, hf[:, None])
    vx, xl, xh, lx = _coords(xs, wf[:, None])

    wy = jnp.stack([1.0 - ly, ly], -1) * vy[..., None].astype(jnp.float32)
    wx = jnp.stack([1.0 - lx, lx], -1) * vx[..., None].astype(jnp.float32)
    ry = base[:, None, None] + jnp.stack([yl, yh], -1) * wi[:, None, None]
    rx = jnp.stack([xl, xh], -1)

    idx = (ry.reshape(n, _P, 1, _S, 2, 1, 1)
           + rx.reshape(n, 1, _P, 1, 1, _S, 2)).reshape(n, _NBINS, _NCONTRIB)
    w = (wy.reshape(n, _P, 1, _S, 2, 1, 1)
         * wx.reshape(n, 1, _P, 1, 1, _S, 2) * (1.0 / (_S * _S))
         ).reshape(n, _NBINS, _NCONTRIB)
    return idx.astype(jnp.int32), w.astype(jnp.float32)


def _sc_gather_pool(table, idx, wbc, n_rois, rois_per_w, nc):
    chunk_rows = _P * _NCONTRIB
    mesh = plsc.VectorSubcoreMesh(core_axis_name="c", subcore_axis_name="s")

    @functools.partial(
        pl.kernel,
        mesh=mesh,
        out_type=jax.ShapeDtypeStruct((n_rois, _NBINS * _C), jnp.float32),
        scratch_types=[
            pltpu.VMEM((_NBINS * _NCONTRIB,), jnp.int32),
            pltpu.VMEM((chunk_rows, _C // 2), jnp.int32),
            pltpu.VMEM((chunk_rows, _C // 2), jnp.int32),
            pltpu.VMEM((_NBINS * _NCONTRIB * 16,), jnp.float32),
            pltpu.VMEM((_NBINS * _C,), jnp.float32),
            pltpu.SemaphoreType.DMA,
            pltpu.SemaphoreType.DMA,
        ],
    )
    def k(table_hbm, idx_hbm, w_hbm, out_hbm,
          idx_v, rows_a, rows_b, w_v, out_v, sem_a, sem_b):
        wid = lax.axis_index("s") * nc + lax.axis_index("c")

        def start(chk, buf, sem):
            pltpu.async_copy(
                table_hbm.at[idx_v.at[pl.ds(chk * chunk_rows, chunk_rows)]],
                buf, sem)

        def wait(buf, sem):
            pltpu.make_async_copy(
                table_hbm.at[pl.ds(0, chunk_rows)], buf, sem).wait()

        def compute(chk, buf):
            def bin_body(b, _):
                bin_g = chk * _P + b
                accs = [jnp.zeros((16,), jnp.float32) for _ in range(_C // 16)]
                for j in range(_NCONTRIB):
                    wv = w_v[pl.ds((bin_g * _NCONTRIB + j) * 16, 16)]
                    row = b * _NCONTRIB + j
                    for g in range(_C // 32):
                        xi = buf[row, pl.ds(g * 16, 16)]
                        lo = lax.bitcast_convert_type(xi << 16, jnp.float32)
                        hi = lax.bitcast_convert_type(
                            xi & jnp.int32(-65536), jnp.float32)
                        accs[2 * g] = accs[2 * g] + wv * lo
                        accs[2 * g + 1] = accs[2 * g + 1] + wv * hi
                for v in range(_C // 16):
                    out_v[pl.ds(bin_g * _C + v * 16, 16)] = accs[v]
                return 0

            lax.fori_loop(0, _P, bin_body, 0)

        def roi_body(i, _):
            r = wid * rois_per_w + i
            pltpu.sync_copy(idx_hbm.at[r], idx_v)
            pltpu.sync_copy(w_hbm.at[r], w_v)
            start(0, rows_a, sem_a)

            def pair_body(p, _):
                start(2 * p + 1, rows_b, sem_b)
                wait(rows_a, sem_a)
                compute(2 * p, rows_a)
                start(2 * p + 2, rows_a, sem_a)
                wait(rows_b, sem_b)
                compute(2 * p + 1, rows_b)
                return 0

            lax.fori_loop(0, (_P - 1) // 2, pair_body, 0)
            wait(rows_a, sem_a)
            compute(_P - 1, rows_a)
            pltpu.sync_copy(out_v, out_hbm.at[r])
            return 0

        lax.fori_loop(0, rois_per_w, roi_body, 0)

    return k(table, idx.reshape(n_rois, _NBINS * _NCONTRIB),
             wbc.reshape(n_rois, _NBINS * _NCONTRIB * 16))


def kernel(feat0, feat1, feat2, feat3, boxes):
    n = boxes.shape[0]
    perm = np.arange(_C).reshape(_C // 32, 2, 16).transpose(0, 2, 1).reshape(_C)
    table = (jnp.concatenate(
        [f.reshape(_C, -1) for f in (feat0, feat1, feat2, feat3)],
        axis=1)[perm].T.astype(jnp.bfloat16))
    table = jax.lax.bitcast_convert_type(
        table.reshape(-1, _C // 2, 2), jnp.int32)

    idx, w = _plan(boxes)
    wbc = jnp.broadcast_to(
        w.reshape(n, _NBINS * _NCONTRIB, 1), (n, _NBINS * _NCONTRIB, 16))

    info = plsc.get_sparse_core_info()
    nw = info.num_cores * info.num_subcores
    rois_per_w = n // nw

    out = _sc_gather_pool(table, idx, wbc, n, rois_per_w, info.num_cores)
    return out.reshape(n, _P, _P, _C).transpose(0, 3, 1, 2)


# speedup vs baseline: 1.5155x; 1.5155x over previous
import functools

import jax
import jax.numpy as jnp
import numpy as np
from jax import lax
from jax.experimental import pallas as pl
from jax.experimental.pallas import tpu as pltpu
from jax.experimental.pallas import tpu_sc as plsc

_P = 7
_S = 2
_C = 256
_SCALES = (0.25, 0.125, 0.0625, 0.03125)
_HWS = ((200, 200), (100, 100), (50, 50), (25, 25))
_NROWS = sum(h * w for h, w in _HWS)
_NBINS = _P * _P
_NCONTRIB = 16


def _coords(c, limit):
    valid = (c >= -1.0) & (c <= limit)
    c = jnp.clip(c, 0.0, None)
    low = jnp.minimum(jnp.floor(c), limit - 1.0)
    high = jnp.minimum(low + 1.0, limit - 1.0)
    c_adj = jnp.where(low >= limit - 1.0, low, c)
    frac = c_adj - low
    return valid, low.astype(jnp.int32), high.astype(jnp.int32), frac


def _plan(boxes):
    n = boxes.shape[0]
    area = (boxes[:, 2] - boxes[:, 0]) * (boxes[:, 3] - boxes[:, 1])
    s = jnp.sqrt(area)
    lvl = jnp.floor(4.0 + jnp.log2(s / 224.0 + 1e-6))
    li = (jnp.clip(lvl, 2.0, 5.0) - 2.0).astype(jnp.int32)

    scale = jnp.asarray(_SCALES, jnp.float32)[li]
    hf = jnp.asarray([h for h, _ in _HWS], jnp.float32)[li]
    wf = jnp.asarray([w for _, w in _HWS], jnp.float32)[li]
    wi = jnp.asarray([w for _, w in _HWS], jnp.int32)[li]
    offs = [0]
    for h, w in _HWS[:-1]:
        offs.append(offs[-1] + h * w)
    base = jnp.asarray(offs, jnp.int32)[li]

    x1 = boxes[:, 0] * scale
    y1 = boxes[:, 1] * scale
    x2 = boxes[:, 2] * scale
    y2 = boxes[:, 3] * scale
    bin_w = jnp.maximum(x2 - x1, 1.0) / _P
    bin_h = jnp.maximum(y2 - y1, 1.0) / _P
    grid = (jnp.arange(_P, dtype=jnp.float32)[:, None]
            + (jnp.arange(_S, dtype=jnp.float32)[None, :] + 0.5) / _S).reshape(-1)
    ys = y1[:, None] + grid[None, :] * bin_h[:, None]
    xs = x1[:, None] + grid[None, :] * bin_w[:, None]
    vy, yl, yh, ly = _coords(ys, hf[:, None])
    vx, xl, xh, lx = _coords(xs, wf[:, None])

    wy = jnp.stack([1.0 - ly, ly], -1) * vy[..., None].astype(jnp.float32)
    wx = jnp.stack([1.0 - lx, lx], -1) * vx[..., None].astype(jnp.float32)
    ry = base[:, None, None] + jnp.stack([yl, yh], -1) * wi[:, None, None]
    rx = jnp.stack([xl, xh], -1)

    idx = (ry.reshape(n, _P, 1, _S, 2, 1, 1)
           + rx.reshape(n, 1, _P, 1, 1, _S, 2)).reshape(n, _NBINS, _NCONTRIB)
    w = (wy.reshape(n, _P, 1, _S, 2, 1, 1)
         * wx.reshape(n, 1, _P, 1, 1, _S, 2) * (1.0 / (_S * _S))
         ).reshape(n, _NBINS, _NCONTRIB)
    return idx.astype(jnp.int32), w.astype(jnp.float32)


def _sc_gather_pool(table, idx, wbc, n_rois, rois_per_w, nc):
    chunk_rows = _P * _NCONTRIB
    mesh = plsc.VectorSubcoreMesh(core_axis_name="c", subcore_axis_name="s")

    @functools.partial(
        pl.kernel,
        mesh=mesh,
        out_type=jax.ShapeDtypeStruct((n_rois, _NBINS * _C), jnp.float32),
        scratch_types=[
            pltpu.VMEM((_NBINS * _NCONTRIB,), jnp.int32),
            pltpu.VMEM((chunk_rows, _C // 2), jnp.int32),
            pltpu.VMEM((chunk_rows, _C // 2), jnp.int32),
            pltpu.VMEM((_NBINS * _NCONTRIB * 16,), jnp.float32),
            pltpu.VMEM((_NBINS * _C,), jnp.float32),
            pltpu.SemaphoreType.DMA,
            pltpu.SemaphoreType.DMA,
        ],
    )
    def k(table_hbm, idx_hbm, w_hbm, out_hbm,
          idx_v, rows_a, rows_b, w_v, out_v, sem_a, sem_b):
        wid = lax.axis_index("s") * nc + lax.axis_index("c")

        def start(chk, buf, sem):
            pltpu.async_copy(
                table_hbm.at[idx_v.at[pl.ds(chk * chunk_rows, chunk_rows)]],
                buf, sem)

        def wait(buf, sem):
            pltpu.make_async_copy(
                table_hbm.at[pl.ds(0, chunk_rows)], buf, sem).wait()

        def compute(chk, buf):
            def bin_body(b, _):
                bin_g = chk * _P + b
                accs = [jnp.zeros((16,), jnp.float32) for _ in range(_C // 16)]
                for j in range(_NCONTRIB):
                    wv = w_v[pl.ds((bin_g * _NCONTRIB + j) * 16, 16)]
                    row = b * _NCONTRIB + j
                    for g in range(_C // 32):
                        xi = buf[row, pl.ds(g * 16, 16)]
                        lo = lax.bitcast_convert_type(xi << 16, jnp.float32)
                        hi = lax.bitcast_convert_type(
                            xi & jnp.int32(-65536), jnp.float32)
                        accs[2 * g] = accs[2 * g] + wv * lo
                        accs[2 * g + 1] = accs[2 * g + 1] + wv * hi
                for v in range(_C // 16):
                    out_v[pl.ds(bin_g * _C + v * 16, 16)] = accs[v]
                return 0

            lax.fori_loop(0, _P, bin_body, 0)

        def roi_body(i, _):
            r = wid * rois_per_w + i
            pltpu.sync_copy(idx_hbm.at[r], idx_v)
            pltpu.sync_copy(w_hbm.at[r], w_v)
            start(0, rows_a, sem_a)

            def pair_body(p, _):
                start(2 * p + 1, rows_b, sem_b)
                wait(rows_a, sem_a)
                compute(2 * p, rows_a)
                start(2 * p + 2, rows_a, sem_a)
                wait(rows_b, sem_b)
                compute(2 * p + 1, rows_b)
                return 0

            lax.fori_loop(0, (_P - 1) // 2, pair_body, 0)
            wait(rows_a, sem_a)
            compute(_P - 1, rows_a)
            pltpu.sync_copy(out_v, out_hbm.at[r])
            return 0

        lax.fori_loop(0, rois_per_w, roi_body, 0)

    return k(table, idx.reshape(n_rois, _NBINS * _NCONTRIB),
             wbc.reshape(n_rois, _NBINS * _NCONTRIB * 16))


def kernel(feat0, feat1, feat2, feat3, boxes):
    n = boxes.shape[0]
    ks = np.arange(_C // 2)
    lo_ch = (ks // 16) * 32 + ks % 16
    hi_ch = lo_ch + 16

    def _pack_level(f):
        ff = f.reshape(_C, -1)
        lo = lax.bitcast_convert_type(
            ff[lo_ch].astype(jnp.bfloat16), jnp.uint16).astype(jnp.uint32)
        hi = lax.bitcast_convert_type(
            ff[hi_ch].astype(jnp.bfloat16), jnp.uint16).astype(jnp.uint32)
        return lax.bitcast_convert_type(lo | (hi << 16), jnp.int32).T

    table = jnp.concatenate(
        [_pack_level(f) for f in (feat0, feat1, feat2, feat3)], axis=0)

    idx, w = _plan(boxes)
    wbc = jnp.broadcast_to(
        w.reshape(n, _NBINS * _NCONTRIB, 1), (n, _NBINS * _NCONTRIB, 16))

    info = plsc.get_sparse_core_info()
    nw = info.num_cores * info.num_subcores
    rois_per_w = n // nw

    out = _sc_gather_pool(table, idx, wbc, n, rois_per_w, info.num_cores)
    return out.reshape(n, _P, _P, _C).transpose(0, 3, 1, 2)


# speedup vs baseline: 1.5598x; 1.0292x over previous
import functools

import jax
import jax.numpy as jnp
import numpy as np
from jax import lax
from jax.experimental import pallas as pl
from jax.experimental.pallas import tpu as pltpu
from jax.experimental.pallas import tpu_sc as plsc

_P = 7
_S = 2
_C = 256
_SCALES = (0.25, 0.125, 0.0625, 0.03125)
_HWS = ((200, 200), (100, 100), (50, 50), (25, 25))
_NROWS = sum(h * w for h, w in _HWS)
_NBINS = _P * _P
_NCONTRIB = 16


def _coords(c, limit):
    valid = (c >= -1.0) & (c <= limit)
    c = jnp.clip(c, 0.0, None)
    low = jnp.minimum(jnp.floor(c), limit - 1.0)
    high = jnp.minimum(low + 1.0, limit - 1.0)
    c_adj = jnp.where(low >= limit - 1.0, low, c)
    frac = c_adj - low
    return valid, low.astype(jnp.int32), high.astype(jnp.int32), frac


def _plan(boxes):
    n = boxes.shape[0]
    area = (boxes[:, 2] - boxes[:, 0]) * (boxes[:, 3] - boxes[:, 1])
    s = jnp.sqrt(area)
    lvl = jnp.floor(4.0 + jnp.log2(s / 224.0 + 1e-6))
    li = (jnp.clip(lvl, 2.0, 5.0) - 2.0).astype(jnp.int32)

    scale = jnp.asarray(_SCALES, jnp.float32)[li]
    hf = jnp.asarray([h for h, _ in _HWS], jnp.float32)[li]
    wf = jnp.asarray([w for _, w in _HWS], jnp.float32)[li]
    wi = jnp.asarray([w for _, w in _HWS], jnp.int32)[li]
    offs = [0]
    for h, w in _HWS[:-1]:
        offs.append(offs[-1] + h * w)
    base = jnp.asarray(offs, jnp.int32)[li]

    x1 = boxes[:, 0] * scale
    y1 = boxes[:, 1] * scale
    x2 = boxes[:, 2] * scale
    y2 = boxes[:, 3] * scale
    bin_w = jnp.maximum(x2 - x1, 1.0) / _P
    bin_h = jnp.maximum(y2 - y1, 1.0) / _P
    grid = (jnp.arange(_P, dtype=jnp.float32)[:, None]
            + (jnp.arange(_S, dtype=jnp.float32)[None, :] + 0.5) / _S).reshape(-1)
    ys = y1[:, None] + grid[None, :] * bin_h[:, None]
    xs = x1[:, None] + grid[None, :] * bin_w[:, None]
    vy, yl, yh, ly = _coords(ys, hf[:, None])
    vx, xl, xh, lx = _coords(xs, wf[:, None])

    wy = jnp.stack([1.0 - ly, ly], -1) * vy[..., None].astype(jnp.float32)
    wx = jnp.stack([1.0 - lx, lx], -1) * vx[..., None].astype(jnp.float32)
    ry = base[:, None, None] + jnp.stack([yl, yh], -1) * wi[:, None, None]
    rx = jnp.stack([xl, xh], -1)

    idx = (ry.reshape(n, _P, 1, _S, 2, 1, 1)
           + rx.reshape(n, 1, _P, 1, 1, _S, 2)).reshape(n, _NBINS, _NCONTRIB)
    w = (wy.reshape(n, _P, 1, _S, 2, 1, 1)
         * wx.reshape(n, 1, _P, 1, 1, _S, 2) * (1.0 / (_S * _S))
         ).reshape(n, _NBINS, _NCONTRIB)
    return idx.astype(jnp.int32), w.astype(jnp.float32)


def _sc_gather_pool(table, idx, wbc, n_rois, rois_per_w, nc):
    chunk_rows = _P * _NCONTRIB
    mesh = plsc.VectorSubcoreMesh(core_axis_name="c", subcore_axis_name="s")

    @functools.partial(
        pl.kernel,
        mesh=mesh,
        out_type=jax.ShapeDtypeStruct((n_rois, _NBINS * _C), jnp.float32),
        scratch_types=[
            pltpu.VMEM((_NBINS * _NCONTRIB,), jnp.int32),
            pltpu.VMEM((chunk_rows, _C // 2), jnp.int32),
            pltpu.VMEM((chunk_rows, _C // 2), jnp.int32),
            pltpu.VMEM((_NBINS * _NCONTRIB * 16,), jnp.float32),
            pltpu.VMEM((_NBINS * _C,), jnp.float32),
            pltpu.SemaphoreType.DMA,
            pltpu.SemaphoreType.DMA,
        ],
    )
    def k(table_hbm, idx_hbm, w_hbm, out_hbm,
          idx_v, rows_a, rows_b, w_v, out_v, sem_a, sem_b):
        wid = lax.axis_index("s") * nc + lax.axis_index("c")

        def start(chk, buf, sem):
            pltpu.async_copy(
                table_hbm.at[idx_v.at[pl.ds(chk * chunk_rows, chunk_rows)]],
                buf, sem)

        def wait(buf, sem):
            pltpu.make_async_copy(
                table_hbm.at[pl.ds(0, chunk_rows)], buf, sem).wait()

        def compute(chk, buf):
            def bin_body(b, _):
                bin_g = chk * _P + b
                accs = [jnp.zeros((16,), jnp.float32) for _ in range(_C // 16)]
                for j in range(_NCONTRIB):
                    wv = w_v[pl.ds((bin_g * _NCONTRIB + j) * 16, 16)]
                    row = b * _NCONTRIB + j
                    for g in range(_C // 32):
                        xi = buf[row, pl.ds(g * 16, 16)]
                        lo = lax.bitcast_convert_type(xi << 16, jnp.float32)
                        hi = lax.bitcast_convert_type(xi, jnp.float32)
                        accs[2 * g] = accs[2 * g] + wv * lo
                        accs[2 * g + 1] = accs[2 * g + 1] + wv * hi
                for v in range(_C // 16):
                    out_v[pl.ds(bin_g * _C + v * 16, 16)] = accs[v]
                return 0

            lax.fori_loop(0, _P, bin_body, 0)

        def roi_body(i, _):
            r = wid * rois_per_w + i
            pltpu.sync_copy(idx_hbm.at[r], idx_v)
            pltpu.sync_copy(w_hbm.at[r], w_v)
            start(0, rows_a, sem_a)

            def pair_body(p, _):
                start(2 * p + 1, rows_b, sem_b)
                wait(rows_a, sem_a)
                compute(2 * p, rows_a)
                start(2 * p + 2, rows_a, sem_a)
                wait(rows_b, sem_b)
                compute(2 * p + 1, rows_b)
                return 0

            lax.fori_loop(0, (_P - 1) // 2, pair_body, 0)
            wait(rows_a, sem_a)
            compute(_P - 1, rows_a)
            pltpu.sync_copy(out_v, out_hbm.at[r])
            return 0

        lax.fori_loop(0, rois_per_w, roi_body, 0)

    return k(table, idx.reshape(n_rois, _NBINS * _NCONTRIB),
             wbc.reshape(n_rois, _NBINS * _NCONTRIB * 16))


def kernel(feat0, feat1, feat2, feat3, boxes):
    n = boxes.shape[0]
    def _pack_level(f):
        fh = f.reshape(_C // 32, 2, 16, -1)
        lo = lax.bitcast_convert_type(
            fh[:, 0].astype(jnp.bfloat16), jnp.uint16).astype(jnp.uint32)
        hi = lax.bitcast_convert_type(
            fh[:, 1].astype(jnp.bfloat16), jnp.uint16).astype(jnp.uint32)
        return lax.bitcast_convert_type(
            (lo | (hi << 16)).reshape(_C // 2, -1), jnp.int32).T

    table = jnp.concatenate(
        [_pack_level(f) for f in (feat0, feat1, feat2, feat3)], axis=0)

    idx, w = _plan(boxes)
    wbc = jnp.broadcast_to(
        w.reshape(n, _NBINS * _NCONTRIB, 1), (n, _NBINS * _NCONTRIB, 16))

    info = plsc.get_sparse_core_info()
    nw = info.num_cores * info.num_subcores
    rois_per_w = n // nw

    out = _sc_gather_pool(table, idx, wbc, n, rois_per_w, info.num_cores)
    return out.reshape(n, _P, _P, _C).transpose(0, 3, 1, 2)
